# baseline (speedup 1.0000x reference)
import functools

import jax
import jax.numpy as jnp
from jax import lax
from jax.experimental import pallas as pl
from jax.experimental.pallas import tpu as pltpu
from jax.experimental.pallas import tpu_sc as plsc

N_NODES = 10000
D = 128
N_EDGES = 320000

NC = 2
NS = 16

B = 128
NBUF = 2
NB = 160
STAGES = ((0, 56), (56, 56), (112, 48))
SB = 56
NB_TOTAL = NS * NB
E_PAD = NB_TOTAL * B

ACC_ROWS = 10240
TRASH = N_NODES
ROWS_PER_TILE = ACC_ROWS // NS
OUT_CHUNKS = ROWS_PER_TILE // B


def _sc_aggregate():
    mesh = plsc.VectorSubcoreMesh(core_axis_name="c", subcore_axis_name="s")

    @functools.partial(
        pl.kernel,
        mesh=mesh,
        out_type=jax.ShapeDtypeStruct((ACC_ROWS, D), jnp.float32),
        scratch_types=[
            pltpu.VMEM((SB, B), jnp.int32),
            pltpu.VMEM((SB, B), jnp.int32),
            pltpu.VMEM((NBUF, B, D), jnp.float32),
            pltpu.VMEM_SHARED((ACC_ROWS, D), jnp.float32),
            pltpu.SemaphoreType.DMA,
            pltpu.SemaphoreType.DMA,
        ],
    )
    def agg(feat_hbm, src_hbm, dst_hbm, out_hbm,
            src_v, dst_v, rows_v, acc, sem0, sem1):
        c = lax.axis_index("c")
        s = lax.axis_index("s")
        sems = (sem0, sem1)

        def gather_start(it, bf):
            pltpu.make_async_copy(
                feat_hbm.at[src_v.at[it]], rows_v.at[bf], sems[bf]).start()

        def run_stage(src_row, dst_row, off, nb):
            pltpu.sync_copy(
                src_row.at[pl.ds(off, nb)], src_v.at[pl.ds(0, nb)])
            pltpu.sync_copy(
                dst_row.at[pl.ds(off, nb)], dst_v.at[pl.ds(0, nb)])
            for bf in range(NBUF):
                gather_start(bf, bf)

            def body(g, carry):
                for bf in range(NBUF):
                    it = g * NBUF + bf
                    pltpu.make_async_copy(
                        feat_hbm.at[src_v.at[it]], rows_v.at[bf],
                        sems[bf]).wait()
                    pltpu.sync_copy(
                        rows_v.at[bf], acc.at[dst_v.at[it]], add=True)

                    @pl.when(it + NBUF < nb)
                    def _():
                        gather_start(it + NBUF, bf)
                return carry

            lax.fori_loop(0, nb // NBUF, body, 0)

        @pl.when(c == 0)
        def _():
            zrow = rows_v.at[0]
            nvec = D // 16

            def zstore(i, carry):
                zrow[i // nvec, pl.ds((i % nvec) * 16, 16)] = jnp.zeros(
                    (16,), jnp.float32)
                return carry

            lax.fori_loop(0, B * nvec, zstore, 0)
            for k in range(OUT_CHUNKS):
                pltpu.sync_copy(
                    zrow, acc.at[pl.ds(s * ROWS_PER_TILE + k * B, B)])

        plsc.subcore_barrier()

        @pl.when(c == 0)
        def _():
            for off, nb in STAGES:
                run_stage(src_hbm.at[s], dst_hbm.at[s], off, nb)

        plsc.subcore_barrier()

        @pl.when(c == 0)
        def _():
            for k in range(OUT_CHUNKS):
                r = s * ROWS_PER_TILE + k * B
                pltpu.sync_copy(acc.at[pl.ds(r, B)], rows_v.at[0])
                pltpu.sync_copy(rows_v.at[0], out_hbm.at[pl.ds(r, B)])

    return agg


_AGG = _sc_aggregate()


def kernel(feat, edge_index, W, b):
    src = edge_index[0].astype(jnp.int32)
    dst = edge_index[1].astype(jnp.int32)
    pad = E_PAD - N_EDGES
    src_p = jnp.concatenate(
        [src, jnp.zeros((pad,), jnp.int32)]).reshape(NS, NB, B)
    dst_p = jnp.concatenate(
        [dst, jnp.full((pad,), TRASH, jnp.int32)]).reshape(NS, NB, B)
    full = _AGG(feat, src_p, dst_p)
    return full[:N_NODES]


# speedup vs baseline: 1.0568x; 1.0568x over previous
import functools

import jax
import jax.numpy as jnp
from jax import lax
from jax.experimental import pallas as pl
from jax.experimental.pallas import tpu as pltpu
from jax.experimental.pallas import tpu_sc as plsc

N_NODES = 10000
D = 128
N_EDGES = 320000

NC = 2
NS = 16

B = 128
NBUF = 2
NB = 160
STAGES = ((0, 56), (56, 56), (112, 48))
SB = 56
NB_TOTAL = NS * NB
E_PAD = NB_TOTAL * B

ACC_ROWS = 10240
TRASH = N_NODES
ROWS_PER_TILE = ACC_ROWS // NS
OUT_CHUNKS = ROWS_PER_TILE // B


def _sc_aggregate():
    mesh = plsc.VectorSubcoreMesh(core_axis_name="c", subcore_axis_name="s")

    @functools.partial(
        pl.kernel,
        mesh=mesh,
        out_type=jax.ShapeDtypeStruct((ACC_ROWS, D), jnp.float32),
        scratch_types=[
            pltpu.VMEM((SB, B), jnp.int32),
            pltpu.VMEM((SB, B), jnp.int32),
            pltpu.VMEM((NBUF, B, D), jnp.float32),
            pltpu.VMEM_SHARED((ACC_ROWS, D), jnp.float32),
            pltpu.SemaphoreType.DMA,
            pltpu.SemaphoreType.DMA,
        ],
    )
    def agg(feat_hbm, src_hbm, dst_hbm, out_hbm,
            src_v, dst_v, rows_v, acc, sem0, sem1):
        c = lax.axis_index("c")
        s = lax.axis_index("s")
        sems = (sem0, sem1)

        def gather_start(it, bf):
            pltpu.make_async_copy(
                feat_hbm.at[src_v.at[it]], rows_v.at[bf], sems[bf]).start()

        def run_stage(src_row, dst_row, off, nb):
            pltpu.sync_copy(
                src_row.at[pl.ds(off, nb)], src_v.at[pl.ds(0, nb)])
            pltpu.sync_copy(
                dst_row.at[pl.ds(off, nb)], dst_v.at[pl.ds(0, nb)])
            for bf in range(NBUF):
                gather_start(bf, bf)

            def body(g, carry):
                for bf in range(NBUF):
                    it = g * NBUF + bf
                    pltpu.make_async_copy(
                        feat_hbm.at[src_v.at[it]], rows_v.at[bf],
                        sems[bf]).wait()
                    pltpu.sync_copy(
                        rows_v.at[bf], acc.at[dst_v.at[it]], add=True)

                    @pl.when(it + NBUF < nb)
                    def _():
                        gather_start(it + NBUF, bf)
                return carry

            lax.fori_loop(0, nb // NBUF, body, 0)

        @pl.when(c == 1)
        def _():
            zrow = rows_v.at[0]
            nvec = D // 16

            def zstore(i, carry):
                zrow[i // nvec, pl.ds((i % nvec) * 16, 16)] = jnp.zeros(
                    (16,), jnp.float32)
                return carry

            lax.fori_loop(0, B * nvec, zstore, 0)
            for k in range(OUT_CHUNKS):
                pltpu.sync_copy(
                    zrow, acc.at[pl.ds(s * ROWS_PER_TILE + k * B, B)])

        plsc.subcore_barrier()

        @pl.when(c == 1)
        def _():
            for off, nb in STAGES:
                run_stage(src_hbm.at[s], dst_hbm.at[s], off, nb)

        plsc.subcore_barrier()

        @pl.when(c == 1)
        def _():
            for k in range(OUT_CHUNKS):
                r = s * ROWS_PER_TILE + k * B
                pltpu.sync_copy(acc.at[pl.ds(r, B)], rows_v.at[0])
                pltpu.sync_copy(rows_v.at[0], out_hbm.at[pl.ds(r, B)])

    return agg


_AGG = _sc_aggregate()


def kernel(feat, edge_index, W, b):
    src = edge_index[0].astype(jnp.int32)
    dst = edge_index[1].astype(jnp.int32)
    pad = E_PAD - N_EDGES
    src_p = jnp.concatenate(
        [src, jnp.zeros((pad,), jnp.int32)]).reshape(NS, NB, B)
    dst_p = jnp.concatenate(
        [dst, jnp.full((pad,), TRASH, jnp.int32)]).reshape(NS, NB, B)
    full = _AGG(feat, src_p, dst_p)
    return full[:N_NODES]
